# baseline (speedup 1.0000x reference)
import functools

import jax
import jax.numpy as jnp
from jax import lax
from jax.experimental import pallas as pl
from jax.experimental.pallas import tpu as pltpu
from jax.experimental.pallas import tpu_sc as plsc

BATCH = 16384
DIM = 16
NUM_CORES = 2
NUM_SUBCORES = 16
NUM_WORKERS = NUM_CORES * NUM_SUBCORES
BPW = BATCH // NUM_WORKERS
LANES = 16
GROUPS = BPW // LANES

_mesh = plsc.VectorSubcoreMesh(core_axis_name="c", subcore_axis_name="s")


@functools.partial(
    pl.kernel,
    mesh=_mesh,
    compiler_params=pltpu.CompilerParams(needs_layout_passes=False,
                                         use_tc_tiling_on_sc=False),
    out_type=jax.ShapeDtypeStruct((BATCH,), jnp.float32),
    scratch_types=[
        pltpu.VMEM((BPW,), jnp.int32),
        pltpu.VMEM((BPW,), jnp.int32),
        pltpu.VMEM((BPW, DIM), jnp.float32),
        pltpu.VMEM((BPW, DIM), jnp.float32),
        pltpu.VMEM((BPW,), jnp.float32),
        pltpu.VMEM((LANES * DIM,), jnp.float32),
        pltpu.VMEM((BPW,), jnp.float32),
        pltpu.SemaphoreType.DMA,
        pltpu.SemaphoreType.DMA,
        pltpu.SemaphoreType.DMA,
    ],
)
def _mirt_sc(user_hbm, item_hbm, theta_hbm, a_hbm, b_hbm, out_hbm,
             uidx_v, iidx_v, th_v, a_v, b_v, trans_v, out_v,
             sem_t, sem_a, sem_b):
    wid = lax.axis_index("s") * NUM_CORES + lax.axis_index("c")
    base = wid * BPW

    pltpu.sync_copy(user_hbm.at[pl.ds(base, BPW)], uidx_v)
    pltpu.sync_copy(item_hbm.at[pl.ds(base, BPW)], iidx_v)

    ct = pltpu.async_copy(theta_hbm.at[uidx_v], th_v, sem_t)
    ca = pltpu.async_copy(a_hbm.at[iidx_v], a_v, sem_a)
    cb = pltpu.async_copy(b_hbm.at[iidx_v], b_v, sem_b)
    ct.wait()
    ca.wait()
    cb.wait()

    lane16 = lax.iota(jnp.int32, LANES) * LANES

    def group_body(g, carry):
        base_g = g * LANES
        for j in range(LANES):
            p = th_v[base_g + j] * a_v[base_g + j]
            plsc.store_scatter(trans_v, [lane16 + j], p)
        acc = jnp.zeros((LANES,), jnp.float32)
        for d in range(DIM):
            acc = acc + trans_v[pl.ds(d * LANES, LANES)]
        bv = b_v[pl.ds(base_g, LANES)]
        out_v[pl.ds(base_g, LANES)] = 1.0 / (1.0 + jnp.exp(bv - acc))
        return carry

    lax.fori_loop(0, GROUPS, group_body, 0)

    pltpu.sync_copy(out_v, out_hbm.at[pl.ds(base, BPW)])


def kernel(user, item, theta_w, a_w, b_w):
    return _mirt_sc(user.astype(jnp.int32), item.astype(jnp.int32),
                    theta_w, a_w, jnp.reshape(b_w, (-1,)))
